# Initial kernel scaffold; baseline (speedup 1.0000x reference)
import jax
import jax.numpy as jnp
from jax.experimental import pallas as pl


def kernel(h, adj, price_diff, utilization, W, a, W1, b1, W2, b2):
    raise NotImplementedError("write your pallas kernel here")


# speedup vs baseline: 1.5988x; 1.5988x over previous
import functools

import jax
import jax.numpy as jnp
from jax.experimental import pallas as pl
from jax.experimental.pallas import tpu as pltpu


def _proj_body(h_ref, w_ref, wh_ref, *, n, bm):
    i = pl.program_id(0)
    rows = i * bm + jax.lax.broadcasted_iota(jnp.int32, (bm, 1), 0)
    hw = jnp.dot(h_ref[...], w_ref[...], preferred_element_type=jnp.float32)
    wh_ref[...] = jnp.where(rows < n, hw, 0.0)


def _attn_body(wh_ref, adj_ref, pd_ref, ut_ref, asrc_ref, adst_ref,
               w1m_ref, w1pd_ref, w1ut_ref, b1_ref, w2_ref, b2_ref,
               out_ref,
               acc_ref, l_ref, pda_ref, uta_ref, m_ref, src_ref,
               *, n, bm, bn, nj):
    i = pl.program_id(0)
    j = pl.program_id(1)

    @pl.when(j == 0)
    def _init():
        acc_ref[...] = jnp.zeros_like(acc_ref)
        l_ref[...] = jnp.zeros_like(l_ref)
        pda_ref[...] = jnp.zeros_like(pda_ref)
        uta_ref[...] = jnp.zeros_like(uta_ref)
        wh_i = wh_ref[pl.ds(i * bm, bm), :]
        srcv = jnp.dot(wh_i, asrc_ref[...], preferred_element_type=jnp.float32)
        dfull = jax.lax.dot_general(
            adst_ref[...], wh_ref[...], (((1,), (1,)), ((), ())),
            preferred_element_type=jnp.float32)
        dmax = jnp.max(dfull)
        sm = srcv + dmax
        m_ref[...] = jnp.maximum(sm, 0.2 * sm)
        src_ref[...] = srcv

    src = src_ref[...]
    m = m_ref[...]
    wh_j = wh_ref[pl.ds(j * bn, bn), :]
    dst = jax.lax.dot_general(
        adst_ref[...], wh_j, (((1,), (1,)), ((), ())),
        preferred_element_type=jnp.float32)
    e = src + dst
    e = jnp.maximum(e, 0.2 * e)
    p = jnp.exp(e - m)
    cols = j * bn + jax.lax.broadcasted_iota(jnp.int32, (1, bn), 1)
    colmask = cols < n
    p = jnp.where((adj_ref[...] > 0.5) & colmask, p, 0.0)
    acc_ref[...] += jnp.dot(p, wh_j, preferred_element_type=jnp.float32)
    l_ref[...] += jnp.sum(p, axis=1, keepdims=True)
    pdt = jnp.where(colmask, pd_ref[...], 0.0)
    utt = jnp.where(colmask, ut_ref[...], 0.0)
    pda_ref[...] += jnp.sum(p * pdt, axis=1, keepdims=True)
    uta_ref[...] += jnp.sum(p * utt, axis=1, keepdims=True)

    @pl.when(j == nj - 1)
    def _fin():
        l = l_ref[...]
        inv = jnp.where(l > 0, 1.0, 0.0) / jnp.where(l > 0, l, 1.0)
        msg = acc_ref[...] * inv
        pdv = pda_ref[...] * inv
        utv = uta_ref[...] * inv
        z = jnp.dot(msg, w1m_ref[...], preferred_element_type=jnp.float32)
        z = z + pdv * w1pd_ref[...] + utv * w1ut_ref[...] + b1_ref[...]
        zr = jnp.maximum(z, 0.0)
        out = jnp.dot(zr, w2_ref[...], preferred_element_type=jnp.float32)
        out_ref[...] = out + b2_ref[...]


def kernel(h, adj, price_diff, utilization, W, a, W1, b1, W2, b2):
    n, d = h.shape
    bm = 400
    bn = 1024
    nj = -(-n // bn)
    npad = nj * bn
    ni = -(-n // bm)

    bm2 = 512
    wh = pl.pallas_call(
        functools.partial(_proj_body, n=n, bm=bm2),
        grid=(npad // bm2,),
        in_specs=[
            pl.BlockSpec((bm2, d), lambda i: (i, 0)),
            pl.BlockSpec((d, d), lambda i: (0, 0)),
        ],
        out_specs=pl.BlockSpec((bm2, d), lambda i: (i, 0)),
        out_shape=jax.ShapeDtypeStruct((npad, d), jnp.float32),
    )(h, W)

    a_src = a[:d].reshape(d, 1)
    a_dst = a[d:].reshape(1, d)
    w1m = W1[:d]
    w1pd = W1[d:d + 1]
    w1ut = W1[d + 1:d + 2]
    b1r = b1.reshape(1, d)
    b2r = b2.reshape(1, d)

    out = pl.pallas_call(
        functools.partial(_attn_body, n=n, bm=bm, bn=bn, nj=nj),
        grid=(ni, nj),
        in_specs=[
            pl.BlockSpec((npad, d), lambda i, j: (0, 0)),
            pl.BlockSpec((bm, bn), lambda i, j: (i, j)),
            pl.BlockSpec((bm, bn), lambda i, j: (i, j)),
            pl.BlockSpec((bm, bn), lambda i, j: (i, j)),
            pl.BlockSpec((d, 1), lambda i, j: (0, 0)),
            pl.BlockSpec((1, d), lambda i, j: (0, 0)),
            pl.BlockSpec((d, d), lambda i, j: (0, 0)),
            pl.BlockSpec((1, d), lambda i, j: (0, 0)),
            pl.BlockSpec((1, d), lambda i, j: (0, 0)),
            pl.BlockSpec((1, d), lambda i, j: (0, 0)),
            pl.BlockSpec((d, d), lambda i, j: (0, 0)),
            pl.BlockSpec((1, d), lambda i, j: (0, 0)),
        ],
        out_specs=pl.BlockSpec((bm, d), lambda i, j: (i, 0)),
        out_shape=jax.ShapeDtypeStruct((n, d), jnp.float32),
        scratch_shapes=[
            pltpu.VMEM((bm, d), jnp.float32),
            pltpu.VMEM((bm, 1), jnp.float32),
            pltpu.VMEM((bm, 1), jnp.float32),
            pltpu.VMEM((bm, 1), jnp.float32),
            pltpu.VMEM((bm, 1), jnp.float32),
            pltpu.VMEM((bm, 1), jnp.float32),
        ],
        compiler_params=pltpu.CompilerParams(
            dimension_semantics=("parallel", "arbitrary")),
    )(wh, adj, price_diff, utilization, a_src, a_dst,
      w1m, w1pd, w1ut, b1r, W2, b2r)
    return out
